# Initial kernel scaffold; baseline (speedup 1.0000x reference)
import jax
import jax.numpy as jnp
from jax.experimental import pallas as pl


def kernel(x, weight):
    raise NotImplementedError("write your pallas kernel here")


# speedup vs baseline: 1.6620x; 1.6620x over previous
import functools

import jax
import jax.numpy as jnp
from jax import lax
from jax.experimental import pallas as pl
from jax.experimental.pallas import tpu as pltpu
from jax.experimental.pallas import tpu_sc as plsc

NUM_EMBEDD = 100000
EMBEDD_DIM = 128
MAX_NORM = 1.0

_NC = 2
_NS = 16
_NW = _NC * _NS
_B = 4096 * 50
_PER_W = _B // _NW
_CHUNK = 128
_NCHUNK = _PER_W // _CHUNK


def _rsqrt_newton(s):
    i = lax.bitcast_convert_type(s, jnp.int32)
    y = lax.bitcast_convert_type(jnp.int32(0x5F3759DF) - (i >> 1), jnp.float32)
    for _ in range(3):
        y = y * (jnp.float32(1.5) - jnp.float32(0.5) * s * y * y)
    return y


def _sc_body(x_hbm, w_hbm, out_hbm, idx_v, rows_v, sem):
    wid = lax.axis_index("s") * _NC + lax.axis_index("c")
    base = wid * _PER_W

    pltpu.sync_copy(x_hbm.at[wid], idx_v)

    def chunk_body(j, carry):
        pltpu.async_copy(w_hbm.at[idx_v.at[j]], rows_v, sem).wait()

        def row_body(r, c):
            vs = [rows_v[r, pl.ds(16 * k, 16)] for k in range(8)]
            acc = vs[0] * vs[0]
            for k in range(1, 8):
                acc = acc + vs[k] * vs[k]
            lanes = lax.iota(jnp.int32, 16)
            dnums = lax.GatherDimensionNumbers(
                offset_dims=(), collapsed_slice_dims=(0,), start_index_map=(0,))
            s = acc
            for sh in (8, 4, 2, 1):
                perm = (lanes ^ sh)[:, None]
                s = s + lax.gather(
                    s, perm, dnums, slice_sizes=(1,),
                    mode=lax.GatherScatterMode.PROMISE_IN_BOUNDS)
            y = _rsqrt_newton(s)
            norm = s * y
            scale = jnp.where(s > jnp.float32(MAX_NORM),
                              jnp.float32(MAX_NORM) / (norm + jnp.float32(1e-7)),
                              jnp.float32(1.0))
            for k in range(8):
                rows_v[r, pl.ds(16 * k, 16)] = vs[k] * scale
            return c

        lax.fori_loop(0, _CHUNK, row_body, 0)
        pltpu.sync_copy(rows_v, out_hbm.at[pl.ds(base + j * _CHUNK, _CHUNK)])
        return carry

    lax.fori_loop(0, _NCHUNK, chunk_body, 0)


@jax.jit
def _run(x2d, weight):
    mesh = plsc.VectorSubcoreMesh(core_axis_name="c", subcore_axis_name="s",
                                  num_cores=_NC, num_subcores=_NS)
    f = pl.kernel(
        _sc_body,
        out_type=jax.ShapeDtypeStruct((_B, EMBEDD_DIM), jnp.float32),
        mesh=mesh,
        scratch_types=[
            pltpu.VMEM((_NCHUNK, _CHUNK), jnp.int32),
            pltpu.VMEM((_CHUNK, EMBEDD_DIM), jnp.float32),
            pltpu.SemaphoreType.DMA,
        ],
    )
    return f(x2d, weight)


def kernel(x, weight):
    x2d = x.astype(jnp.int32).reshape(_NW, _NCHUNK, _CHUNK)
    out = _run(x2d, weight)
    return out.reshape(x.shape[0], x.shape[1], EMBEDD_DIM)
